# baseline (speedup 1.0000x reference)
import functools

import jax
import jax.numpy as jnp
from jax import lax
from jax.experimental import pallas as pl
from jax.experimental.pallas import tpu as pltpu
from jax.experimental.pallas import tpu_sc as plsc

N = 10000
E = 320000
D = 128
NC = 2
NS = 16
NW = NC * NS
DH = D // NC
EPT = E // NW
BB = 100
NB = EPT // BB
EPS = E // NS
NB2 = EPS // BB
NPAD = 10240
RPT = NPAD // NS
RCH = 80
DPT = NPAD // NS

def _sc_degree(dst_hbm, out_hbm, dstv, ones, stage, acc):
    cid = lax.axis_index("c")
    sid = lax.axis_index("s")
    wid = sid * NC + cid
    pltpu.sync_copy(dst_hbm.at[wid], dstv)
    for k in range(112 // 16):
        ones[pl.ds(k * 16, 16)] = jnp.ones((16,), jnp.float32)
    for k in range(DPT // 16):
        stage[pl.ds(k * 16, 16)] = jnp.zeros((16,), jnp.float32)
    pltpu.sync_copy(stage, acc.at[pl.ds(sid * DPT, DPT)])
    plsc.subcore_barrier()

    def body(j, carry):
        pltpu.sync_copy(ones.at[pl.ds(0, BB)], acc.at[dstv.at[j]], add=True)
        return carry

    lax.fori_loop(0, NB, body, 0)
    plsc.subcore_barrier()
    pltpu.sync_copy(acc.at[pl.ds(sid * DPT, DPT)], stage)
    pltpu.sync_copy(stage, out_hbm.at[cid, pl.ds(sid * DPT, DPT)])


NBUF = 4


def _sc_gather_scatter(hp_hbm, src_hbm, dst_hbm, out_hbm,
                       srcv, dstv, rows0, rows1, rows2, rows3, acc,
                       sem0, sem1, sem2, sem3,
                       ssem0, ssem1, ssem2, ssem3):
    cid = lax.axis_index("c")
    sid = lax.axis_index("s")
    hpc = hp_hbm.at[cid]
    rows = (rows0, rows1, rows2, rows3)
    sems = (sem0, sem1, sem2, sem3)
    ssems = (ssem0, ssem1, ssem2, ssem3)
    pltpu.sync_copy(src_hbm.at[sid], srcv)
    pltpu.sync_copy(dst_hbm.at[sid], dstv)

    def zrow(i, carry):
        for k in range(DH // 16):
            rows0[i, pl.ds(k * 16, 16)] = jnp.zeros((16,), jnp.float32)
        return carry

    lax.fori_loop(0, RCH, zrow, 0)
    zslice = rows0.at[pl.ds(0, RCH)]
    for k in range(RPT // RCH):
        pltpu.sync_copy(zslice, acc.at[pl.ds(sid * RPT + k * RCH, RCH)])
    plsc.subcore_barrier()

    for b in range(NBUF - 1):
        pltpu.async_copy(hpc.at[srcv.at[b]], rows[b], sems[b])

    def step(j, b, prefetch, first=False):
        pltpu.make_async_copy(hpc.at[srcv.at[j]], rows[b], sems[b]).wait()
        bp = (b + NBUF - 1) % NBUF

        def _wait_prev():
            pltpu.make_async_copy(
                rows[bp], acc.at[dstv.at[j - 1]], ssems[bp]).wait()

        if first:
            pl.when(j >= 1)(_wait_prev)
        else:
            _wait_prev()
        if prefetch:
            pltpu.async_copy(
                hpc.at[srcv.at[j + NBUF - 1]], rows[bp], sems[bp])
        pltpu.async_copy(rows[b], acc.at[dstv.at[j]], ssems[b], add=True)

    def body(j4, carry):
        for b in range(NBUF):
            step(j4 * NBUF + b, b, True, first=(b == 0))
        return carry

    lax.fori_loop(0, NB2 // NBUF - 1, body, 0)
    for b in range(NBUF):
        step(NB2 - NBUF + b, b, b == 0)
    pltpu.make_async_copy(
        rows[NBUF - 1], acc.at[dstv.at[NB2 - 1]], ssems[NBUF - 1]).wait()
    plsc.subcore_barrier()
    for k in range(RPT // RCH):
        off = sid * RPT + k * RCH
        pltpu.sync_copy(acc.at[pl.ds(off, RCH)], zslice)
        pltpu.sync_copy(zslice, out_hbm.at[cid, pl.ds(off, RCH)])


@functools.lru_cache(maxsize=None)
def _sc_kernels(interpret=False):
    mesh = plsc.VectorSubcoreMesh(
        core_axis_name="c", subcore_axis_name="s",
        num_cores=NC, num_subcores=NS)
    params = pltpu.CompilerParams(use_tc_tiling_on_sc=False)
    deg = pl.kernel(
        _sc_degree,
        out_type=jax.ShapeDtypeStruct((NC, NPAD), jnp.float32),
        mesh=mesh,
        compiler_params=params,
        interpret=interpret,
        scratch_types=[
            pltpu.VMEM((NB, BB), jnp.int32),
            pltpu.VMEM((112,), jnp.float32),
            pltpu.VMEM((DPT,), jnp.float32),
            pltpu.VMEM_SHARED((NPAD,), jnp.float32),
        ],
    )
    gs = pl.kernel(
        _sc_gather_scatter,
        out_type=jax.ShapeDtypeStruct((NC, NPAD, DH), jnp.float32),
        mesh=mesh,
        compiler_params=params,
        interpret=interpret,
        scratch_types=[
            pltpu.VMEM((NB2, BB), jnp.int32),
            pltpu.VMEM((NB2, BB), jnp.int32),
            pltpu.VMEM((BB, DH), jnp.float32),
            pltpu.VMEM((BB, DH), jnp.float32),
            pltpu.VMEM((BB, DH), jnp.float32),
            pltpu.VMEM((BB, DH), jnp.float32),
            pltpu.VMEM_SHARED((NPAD, DH), jnp.float32),
            pltpu.SemaphoreType.DMA,
            pltpu.SemaphoreType.DMA,
            pltpu.SemaphoreType.DMA,
            pltpu.SemaphoreType.DMA,
            pltpu.SemaphoreType.DMA,
            pltpu.SemaphoreType.DMA,
            pltpu.SemaphoreType.DMA,
            pltpu.SemaphoreType.DMA,
        ],
    )
    return deg, gs


def _dis(degp_ref):
    deg = degp_ref[:, 0:1] + degp_ref[:, 1:2] + 1.0
    return lax.rsqrt(deg)


def _tc_pre_body(x_ref, w_ref, degp_ref, out_ref):
    h = jnp.dot(x_ref[...], w_ref[...],
                preferred_element_type=jnp.float32) * _dis(degp_ref)
    out_ref[0] = h[:, 0:DH]
    out_ref[1] = h[:, DH:D]


def _tc_mid_body(agg_ref, hp_ref, degp_ref, w_ref, b_ref, out_ref):
    dis = _dis(degp_ref)
    z = jnp.concatenate(
        [agg_ref[0, 0:N] + hp_ref[0], agg_ref[1, 0:N] + hp_ref[1]], axis=1)
    z = z * dis + b_ref[...]
    z = jnp.where(z >= 0.0, z, 0.2 * z)
    h = jnp.dot(z, w_ref[...], preferred_element_type=jnp.float32) * dis
    out_ref[0] = h[:, 0:DH]
    out_ref[1] = h[:, DH:D]


def _tc_post_body(agg_ref, hp_ref, degp_ref, b_ref, out_ref):
    z = jnp.concatenate(
        [agg_ref[0, 0:N] + hp_ref[0], agg_ref[1, 0:N] + hp_ref[1]], axis=1)
    z = z * _dis(degp_ref) + b_ref[...]
    out_ref[...] = jnp.where(z >= 0.0, z, 0.2 * z)


_HP_T = jax.ShapeDtypeStruct((NC, N, DH), jnp.float32)


def kernel(X, Adj, W1, b1, W2, b2):
    adj = Adj.astype(jnp.int32)
    dst_deg = adj[1].reshape(NW, NB, BB)
    src3 = adj[0].reshape(NS, NB2, BB)
    dst3 = adj[1].reshape(NS, NB2, BB)
    sc_degree, sc_gather_scatter = _sc_kernels()
    degp = sc_degree(dst_deg)
    degp = jnp.transpose(degp)[:N]
    h1p = pl.pallas_call(_tc_pre_body, out_shape=_HP_T)(X, W1, degp)
    agg1 = sc_gather_scatter(h1p, src3, dst3)
    h2p = pl.pallas_call(_tc_mid_body, out_shape=_HP_T)(
        agg1, h1p, degp, W2, b1.reshape(1, D))
    agg2 = sc_gather_scatter(h2p, src3, dst3)
    return pl.pallas_call(
        _tc_post_body, out_shape=jax.ShapeDtypeStruct((N, D), jnp.float32))(
        agg2, h2p, degp, b2.reshape(1, D))


# speedup vs baseline: 1.0034x; 1.0034x over previous
import functools

import jax
import jax.numpy as jnp
from jax import lax
from jax.experimental import pallas as pl
from jax.experimental.pallas import tpu as pltpu
from jax.experimental.pallas import tpu_sc as plsc

N = 10000
E = 320000
D = 128
NC = 2
NS = 16
NW = NC * NS
DH = D // NC
EPT = E // NW
BB = 125
NB = EPT // BB
EPS = E // NS
NB2 = EPS // BB
NPAD = 10240
RPT = NPAD // NS
RCH = 80
DPT = NPAD // NS

def _sc_degree(dst_hbm, out_hbm, dstv, ones, stage, acc):
    cid = lax.axis_index("c")
    sid = lax.axis_index("s")
    wid = sid * NC + cid
    pltpu.sync_copy(dst_hbm.at[wid], dstv)
    for k in range(128 // 16):
        ones[pl.ds(k * 16, 16)] = jnp.ones((16,), jnp.float32)
    for k in range(DPT // 16):
        stage[pl.ds(k * 16, 16)] = jnp.zeros((16,), jnp.float32)
    pltpu.sync_copy(stage, acc.at[pl.ds(sid * DPT, DPT)])
    plsc.subcore_barrier()

    def body(j, carry):
        pltpu.sync_copy(ones.at[pl.ds(0, BB)], acc.at[dstv.at[j]], add=True)
        return carry

    lax.fori_loop(0, NB, body, 0)
    plsc.subcore_barrier()
    pltpu.sync_copy(acc.at[pl.ds(sid * DPT, DPT)], stage)
    pltpu.sync_copy(stage, out_hbm.at[cid, pl.ds(sid * DPT, DPT)])


NBUF = 4


def _sc_gather_scatter(hp_hbm, src_hbm, dst_hbm, out_hbm,
                       srcv, dstv, rows0, rows1, rows2, rows3, acc,
                       sem0, sem1, sem2, sem3,
                       ssem0, ssem1, ssem2, ssem3):
    cid = lax.axis_index("c")
    sid = lax.axis_index("s")
    hpc = hp_hbm.at[cid]
    rows = (rows0, rows1, rows2, rows3)
    sems = (sem0, sem1, sem2, sem3)
    ssems = (ssem0, ssem1, ssem2, ssem3)
    pltpu.sync_copy(src_hbm.at[sid], srcv)
    pltpu.sync_copy(dst_hbm.at[sid], dstv)

    def zrow(i, carry):
        for k in range(DH // 16):
            rows0[i, pl.ds(k * 16, 16)] = jnp.zeros((16,), jnp.float32)
        return carry

    lax.fori_loop(0, RCH, zrow, 0)
    zslice = rows0.at[pl.ds(0, RCH)]
    for k in range(RPT // RCH):
        pltpu.sync_copy(zslice, acc.at[pl.ds(sid * RPT + k * RCH, RCH)])
    plsc.subcore_barrier()

    for b in range(NBUF - 1):
        pltpu.async_copy(hpc.at[srcv.at[b]], rows[b], sems[b])

    def step(j, b, prefetch, first=False):
        pltpu.make_async_copy(hpc.at[srcv.at[j]], rows[b], sems[b]).wait()
        bp = (b + NBUF - 1) % NBUF

        def _wait_prev():
            pltpu.make_async_copy(
                rows[bp], acc.at[dstv.at[j - 1]], ssems[bp]).wait()

        if first:
            pl.when(j >= 1)(_wait_prev)
        else:
            _wait_prev()
        if prefetch:
            pltpu.async_copy(
                hpc.at[srcv.at[j + NBUF - 1]], rows[bp], sems[bp])
        pltpu.async_copy(rows[b], acc.at[dstv.at[j]], ssems[b], add=True)

    def body(j4, carry):
        for b in range(NBUF):
            step(j4 * NBUF + b, b, True, first=(b == 0))
        return carry

    lax.fori_loop(0, NB2 // NBUF - 1, body, 0)
    for b in range(NBUF):
        step(NB2 - NBUF + b, b, b == 0)
    pltpu.make_async_copy(
        rows[NBUF - 1], acc.at[dstv.at[NB2 - 1]], ssems[NBUF - 1]).wait()
    plsc.subcore_barrier()
    for k in range(RPT // RCH):
        off = sid * RPT + k * RCH
        pltpu.sync_copy(acc.at[pl.ds(off, RCH)], zslice)
        pltpu.sync_copy(zslice, out_hbm.at[cid, pl.ds(off, RCH)])


@functools.lru_cache(maxsize=None)
def _sc_kernels(interpret=False):
    mesh = plsc.VectorSubcoreMesh(
        core_axis_name="c", subcore_axis_name="s",
        num_cores=NC, num_subcores=NS)
    params = pltpu.CompilerParams(use_tc_tiling_on_sc=False)
    deg = pl.kernel(
        _sc_degree,
        out_type=jax.ShapeDtypeStruct((NC, NPAD), jnp.float32),
        mesh=mesh,
        compiler_params=params,
        interpret=interpret,
        scratch_types=[
            pltpu.VMEM((NB, BB), jnp.int32),
            pltpu.VMEM((128,), jnp.float32),
            pltpu.VMEM((DPT,), jnp.float32),
            pltpu.VMEM_SHARED((NPAD,), jnp.float32),
        ],
    )
    gs = pl.kernel(
        _sc_gather_scatter,
        out_type=jax.ShapeDtypeStruct((NC, NPAD, DH), jnp.float32),
        mesh=mesh,
        compiler_params=params,
        interpret=interpret,
        scratch_types=[
            pltpu.VMEM((NB2, BB), jnp.int32),
            pltpu.VMEM((NB2, BB), jnp.int32),
            pltpu.VMEM((BB, DH), jnp.float32),
            pltpu.VMEM((BB, DH), jnp.float32),
            pltpu.VMEM((BB, DH), jnp.float32),
            pltpu.VMEM((BB, DH), jnp.float32),
            pltpu.VMEM_SHARED((NPAD, DH), jnp.float32),
            pltpu.SemaphoreType.DMA,
            pltpu.SemaphoreType.DMA,
            pltpu.SemaphoreType.DMA,
            pltpu.SemaphoreType.DMA,
            pltpu.SemaphoreType.DMA,
            pltpu.SemaphoreType.DMA,
            pltpu.SemaphoreType.DMA,
            pltpu.SemaphoreType.DMA,
        ],
    )
    return deg, gs


def _dis(degp_ref):
    deg = degp_ref[:, 0:1] + degp_ref[:, 1:2] + 1.0
    return lax.rsqrt(deg)


def _tc_pre_body(x_ref, w_ref, degp_ref, out_ref):
    h = jnp.dot(x_ref[...], w_ref[...],
                preferred_element_type=jnp.float32) * _dis(degp_ref)
    out_ref[0] = h[:, 0:DH]
    out_ref[1] = h[:, DH:D]


def _tc_mid_body(agg_ref, hp_ref, degp_ref, w_ref, b_ref, out_ref):
    dis = _dis(degp_ref)
    z = jnp.concatenate(
        [agg_ref[0, 0:N] + hp_ref[0], agg_ref[1, 0:N] + hp_ref[1]], axis=1)
    z = z * dis + b_ref[...]
    z = jnp.where(z >= 0.0, z, 0.2 * z)
    h = jnp.dot(z, w_ref[...], preferred_element_type=jnp.float32) * dis
    out_ref[0] = h[:, 0:DH]
    out_ref[1] = h[:, DH:D]


def _tc_post_body(agg_ref, hp_ref, degp_ref, b_ref, out_ref):
    z = jnp.concatenate(
        [agg_ref[0, 0:N] + hp_ref[0], agg_ref[1, 0:N] + hp_ref[1]], axis=1)
    z = z * _dis(degp_ref) + b_ref[...]
    out_ref[...] = jnp.where(z >= 0.0, z, 0.2 * z)


_HP_T = jax.ShapeDtypeStruct((NC, N, DH), jnp.float32)


def kernel(X, Adj, W1, b1, W2, b2):
    adj = Adj.astype(jnp.int32)
    dst_deg = adj[1].reshape(NW, NB, BB)
    src3 = adj[0].reshape(NS, NB2, BB)
    dst3 = adj[1].reshape(NS, NB2, BB)
    sc_degree, sc_gather_scatter = _sc_kernels()
    degp = sc_degree(dst_deg)
    degp = jnp.transpose(degp)[:N]
    h1p = pl.pallas_call(_tc_pre_body, out_shape=_HP_T)(X, W1, degp)
    agg1 = sc_gather_scatter(h1p, src3, dst3)
    h2p = pl.pallas_call(_tc_mid_body, out_shape=_HP_T)(
        agg1, h1p, degp, W2, b1.reshape(1, D))
    agg2 = sc_gather_scatter(h2p, src3, dst3)
    return pl.pallas_call(
        _tc_post_body, out_shape=jax.ShapeDtypeStruct((N, D), jnp.float32))(
        agg2, h2p, degp, b2.reshape(1, D))
